# Initial kernel scaffold; baseline (speedup 1.0000x reference)
import jax
import jax.numpy as jnp
from jax.experimental import pallas as pl


def kernel(feat, adj, W, bias, prelu_a):
    raise NotImplementedError("write your pallas kernel here")


# speedup vs baseline: 1.0910x; 1.0910x over previous
import functools

import jax
import jax.numpy as jnp
from jax.experimental import pallas as pl
from jax.experimental.pallas import tpu as pltpu


def _h_body(feat_ref, w_ref, h_ref):
    h_ref[...] = jax.lax.dot_general(
        feat_ref[...], w_ref[...],
        dimension_numbers=(((1,), (1,)), ((), ())),
        preferred_element_type=jnp.float32,
    )


def _gcn_body(a_ref, adj_ref, h_ref, bias_ref, out_ref):
    out = jax.lax.dot_general(
        adj_ref[...], h_ref[...],
        dimension_numbers=(((1,), (0,)), ((), ())),
        preferred_element_type=jnp.float32,
    )
    out = out + bias_ref[...]
    alpha = a_ref[0, 0]
    out_ref[...] = jnp.where(out >= 0, out, alpha * out)


@functools.partial(jax.jit, static_argnames=("bm",))
def _gcn(feat2, adj2, W, bias2, a2, bm):
    n, d_in = feat2.shape
    d_out = W.shape[0]

    h = pl.pallas_call(
        _h_body,
        out_shape=jax.ShapeDtypeStruct((n, d_out), jnp.float32),
    )(feat2, W)

    out = pl.pallas_call(
        _gcn_body,
        grid=(n // bm,),
        in_specs=[
            pl.BlockSpec(memory_space=pltpu.SMEM),
            pl.BlockSpec((bm, n), lambda i: (i, 0)),
            pl.BlockSpec((n, d_out), lambda i: (0, 0)),
            pl.BlockSpec((1, d_out), lambda i: (0, 0)),
        ],
        out_specs=pl.BlockSpec((bm, d_out), lambda i: (i, 0)),
        out_shape=jax.ShapeDtypeStruct((n, d_out), jnp.float32),
        compiler_params=pltpu.CompilerParams(
            dimension_semantics=("parallel",),
        ),
    )(a2, adj2, h, bias2)
    return out


def kernel(feat, adj, W, bias, prelu_a):
    b, n, d_in = feat.shape
    d_out = W.shape[0]
    feat2 = feat.reshape(n, d_in)
    adj2 = adj.reshape(n, n)
    bias2 = bias.reshape(1, d_out)
    a2 = jnp.asarray(prelu_a, jnp.float32).reshape(1, 1)
    bm = 256 if n % 256 == 0 else n
    out = _gcn(feat2, adj2, W, bias2, a2, bm)
    return out.reshape(b, n, d_out)
